# Initial kernel scaffold; baseline (speedup 1.0000x reference)
import jax
import jax.numpy as jnp
from jax.experimental import pallas as pl


def kernel(encoding, speaker, edge_index, edge_type, spk_table, bases, comp, root_w, rgcn_bias, gc_w_rel, gc_w_root, gc_bias, W1, b1, W2, b2, W3, b3):
    raise NotImplementedError("write your pallas kernel here")


# speedup vs baseline: 3.5313x; 3.5313x over previous
import functools

import jax
import jax.numpy as jnp
from jax import lax
from jax.experimental import pallas as pl
from jax.experimental.pallas import tpu as pltpu
from jax.experimental.pallas import tpu_sc as plsc

N = 10000
E = 160000
D = 256
R = 16
NB = 8
NC = 2
NS = 16
NW = NC * NS
E_PAD = 163840
EPT = E_PAD // NW
WT = 20 + 1
NBKT = WT * R
E_SORT = 208896
NBLK = E_SORT // 128
EST = E_SORT // NW
DENP = 160128
CHUNK = 128
H1 = 128
H2 = 64

_mesh = plsc.VectorSubcoreMesh(core_axis_name="c", subcore_axis_name="s",
                               num_cores=NC, num_subcores=NS)


def _wid():
    return lax.axis_index("c") * NS + lax.axis_index("s")


@functools.partial(
    pl.kernel,
    out_type=[
        jax.ShapeDtypeStruct((NC * DENP,), jnp.float32),
        jax.ShapeDtypeStruct((E_PAD,), jnp.int32),
        jax.ShapeDtypeStruct((E_PAD,), jnp.int32),
        jax.ShapeDtypeStruct((NW * NBKT,), jnp.int32),
    ],
    mesh=_mesh,
    scratch_types=[
        pltpu.VMEM((CHUNK,), jnp.int32),
        pltpu.VMEM((CHUNK,), jnp.int32),
        pltpu.VMEM((CHUNK,), jnp.int32),
        pltpu.VMEM((CHUNK + 16,), jnp.int32),
        pltpu.VMEM((CHUNK + 16,), jnp.int32),
        pltpu.VMEM((CHUNK,), jnp.float32),
        pltpu.VMEM((NBKT + 16,), jnp.int32),
        pltpu.VMEM((5008,), jnp.float32),
        pltpu.VMEM_SHARED((DENP,), jnp.float32),
        pltpu.SemaphoreType.DMA,
    ],
)
def _sc_edge1(dst_hbm, typ_hbm, den_hbm, bk_hbm, lrk_hbm, hist_hbm,
              dst_v, typ_v, key_v, bkt_v, lrk_v, ones_v, cnt_v, zv,
              den_sp, sem):
    cid = lax.axis_index("c")
    sid = lax.axis_index("s")
    w = cid * NS + sid

    def zb(j, _):
        zv[pl.ds(j * 16, 16)] = jnp.zeros((16,), jnp.float32)
        return _
    lax.fori_loop(0, 5008 // 16, zb, None)

    def zc(j, _):
        cnt_v[pl.ds(j * 16, 16)] = jnp.zeros((16,), jnp.int32)
        return _
    lax.fori_loop(0, (NBKT + 16) // 16, zc, None)

    def ob(j, _):
        ones_v[pl.ds(j * 16, 16)] = jnp.ones((16,), jnp.float32)
        return _
    lax.fori_loop(0, CHUNK // 16, ob, None)

    off = sid * 10008
    pltpu.sync_copy(zv, den_sp.at[pl.ds(off, 5008)])
    pltpu.sync_copy(zv.at[pl.ds(0, 5000)], den_sp.at[pl.ds(off + 5008, 5000)])
    plsc.subcore_barrier()

    tile_base = w * EPT
    lane0 = (lax.broadcasted_iota(jnp.int32, (16,), 0) == 0)

    def chunk(i, _):
        base = tile_base + i * CHUNK
        pltpu.sync_copy(dst_hbm.at[pl.ds(base, CHUNK)], dst_v)
        pltpu.sync_copy(typ_hbm.at[pl.ds(base, CHUNK)], typ_v)

        def keyb(j, _):
            o = j * 16
            d = dst_v[pl.ds(o, 16)]
            t = typ_v[pl.ds(o, 16)]
            key_v[pl.ds(o, 16)] = jnp.minimum(d, N) * 16 + t
            bkt_v[pl.ds(o, 16)] = ((d >> 9) << 4) + t
            return _
        lax.fori_loop(0, CHUNK // 16, keyb, None)
        pltpu.sync_copy(ones_v, den_sp.at[key_v], add=True)
        pltpu.sync_copy(bkt_v.at[pl.ds(0, CHUNK)], bk_hbm.at[pl.ds(base, CHUNK)])

        def rankb(e, _):
            bwin = bkt_v[pl.ds(e, 16)]
            b0 = bwin[0]
            cwin = cnt_v[pl.ds(b0, 16)]
            r0 = cwin[0]
            cnt_v[pl.ds(b0, 16)] = jnp.where(lane0, r0 + 1, cwin)
            lwin = lrk_v[pl.ds(e, 16)]
            lrk_v[pl.ds(e, 16)] = jnp.where(lane0, r0, lwin)
            return _
        lax.fori_loop(0, CHUNK, rankb, None)
        pltpu.sync_copy(lrk_v.at[pl.ds(0, CHUNK)], lrk_hbm.at[pl.ds(base, CHUNK)])
        return _
    lax.fori_loop(0, EPT // CHUNK, chunk, None)

    pltpu.sync_copy(cnt_v.at[pl.ds(0, NBKT)], hist_hbm.at[pl.ds(w * NBKT, NBKT)])
    plsc.subcore_barrier()
    cbase = cid * DENP
    pltpu.sync_copy(den_sp.at[pl.ds(off, 5008)], zv)
    pltpu.sync_copy(zv, den_hbm.at[pl.ds(cbase + off, 5008)])
    pltpu.sync_copy(den_sp.at[pl.ds(off + 5008, 5000)], zv.at[pl.ds(0, 5000)])
    pltpu.sync_copy(zv.at[pl.ds(0, 5000)],
                    den_hbm.at[pl.ds(cbase + off + 5008, 5000)])


@functools.partial(
    pl.kernel,
    out_type=[
        jax.ShapeDtypeStruct((E_SORT,), jnp.int32),
        jax.ShapeDtypeStruct((E_SORT,), jnp.int32),
    ],
    mesh=_mesh,
    scratch_types=[
        pltpu.VMEM((CHUNK,), jnp.int32),
        pltpu.VMEM((CHUNK,), jnp.int32),
        pltpu.VMEM((CHUNK,), jnp.int32),
        pltpu.SemaphoreType.DMA,
    ],
)
def _sc_permute(src_hbm, dst_hbm, pos_hbm, srcs_hbm, dsts_hbm,
                val_v, dst2_v, pos_v, sem):
    w = _wid()
    tile_base = w * EPT

    def chunk(i, _):
        base = tile_base + i * CHUNK
        pltpu.sync_copy(pos_hbm.at[pl.ds(base, CHUNK)], pos_v)
        pltpu.sync_copy(src_hbm.at[pl.ds(base, CHUNK)], val_v)
        pltpu.sync_copy(dst_hbm.at[pl.ds(base, CHUNK)], dst2_v)
        pltpu.async_copy(val_v, srcs_hbm.at[pos_v], sem).wait()
        pltpu.async_copy(dst2_v, dsts_hbm.at[pos_v], sem).wait()
        return _
    lax.fori_loop(0, EPT // CHUNK, chunk, None)


@functools.partial(
    pl.kernel,
    out_type=jax.ShapeDtypeStruct((E_SORT, D), jnp.float32),
    mesh=_mesh,
    scratch_types=[
        pltpu.VMEM((CHUNK,), jnp.int32),
        pltpu.VMEM((CHUNK, D), jnp.float32),
        pltpu.SemaphoreType.DMA,
    ],
)
def _sc_rowgather(tab_hbm, idx_hbm, out_hbm, idx_v, rows_v, sem):
    w = _wid()
    tile_base = w * EST

    def chunk(i, _):
        base = tile_base + i * CHUNK
        pltpu.sync_copy(idx_hbm.at[pl.ds(base, CHUNK)], idx_v)

        def clampb(j, _):
            o = j * 16
            v = idx_v[pl.ds(o, 16)]
            idx_v[pl.ds(o, 16)] = jnp.clip(v, 0, N - 1)
            return _
        lax.fori_loop(0, CHUNK // 16, clampb, None)
        pltpu.async_copy(tab_hbm.at[idx_v], rows_v, sem).wait()
        pltpu.sync_copy(rows_v, out_hbm.at[pl.ds(base, CHUNK)])
        return _
    lax.fori_loop(0, EST // CHUNK, chunk, None)


def _spk_body(spk_ref, tab_ref, o_ref):
    s = spk_ref[...]
    o_ref[...] = jnp.where(s == 0, tab_ref[0:1, :], tab_ref[1:2, :])


def _spk_emb(speaker2d, spk_table):
    return pl.pallas_call(
        _spk_body,
        grid=(10,),
        in_specs=[
            pl.BlockSpec((N // 10, 1), lambda i: (i, 0)),
            pl.BlockSpec((2, 32), lambda i: (0, 0)),
        ],
        out_specs=pl.BlockSpec((N // 10, 32), lambda i: (i, 0)),
        out_shape=jax.ShapeDtypeStruct((N, 32), jnp.float32),
    )(speaker2d, spk_table)


def _invc_body(d_ref, o_ref):
    s = d_ref[0] + d_ref[1]
    o_ref[...] = 1.0 / jnp.maximum(s, 1.0)


def _invc(den):
    return pl.pallas_call(
        _invc_body,
        out_shape=jax.ShapeDtypeStruct((DENP // 128, 128), jnp.float32),
    )(den.reshape(2, DENP // 128, 128)).reshape(DENP)


def _wr_body(comp_ref, bases_ref, o_ref):
    o_ref[...] = jnp.dot(comp_ref[...], bases_ref[...],
                         preferred_element_type=jnp.float32)


def _wr_all(comp, bases):
    out = pl.pallas_call(
        _wr_body,
        out_shape=jax.ShapeDtypeStruct((R, D * D), jnp.float32),
    )(comp, bases.reshape(NB, D * D))
    return out.reshape(R, D, D).astype(jnp.bfloat16)


def _pos_body(bk_ref, lrk_ref, b2_ref, o_ref):
    oh = (bk_ref[...] == lax.broadcasted_iota(jnp.int32, (1, NBKT), 1))
    row = b2_ref[0]
    o_ref[...] = jnp.sum(jnp.where(oh, row, 0), axis=1,
                         keepdims=True) + lrk_ref[...]


def _pos(bk, lrk, base2):
    return pl.pallas_call(
        _pos_body,
        grid=(NW,),
        in_specs=[
            pl.BlockSpec((EPT, 1), lambda i: (i, 0)),
            pl.BlockSpec((EPT, 1), lambda i: (i, 0)),
            pl.BlockSpec((1, 1, NBKT), lambda i: (i, 0, 0)),
        ],
        out_specs=pl.BlockSpec((EPT, 1), lambda i: (i, 0)),
        out_shape=jax.ShapeDtypeStruct((E_PAD, 1), jnp.int32),
    )(bk.reshape(E_PAD, 1), lrk.reshape(E_PAD, 1),
      base2.reshape(NW, 1, NBKT)).reshape(E_PAD)


def _agg1_body(bkt_ref, ub_ref, xg_ref, dst_ref, wr_ref, invc_ref, o_ref):
    i = pl.program_id(0)
    bkt = bkt_ref[i]
    rel = bkt % R
    wt = bkt // R
    prevw = bkt_ref[jnp.maximum(i - 1, 0)] // R
    first = jnp.logical_or(i == 0, wt != prevw)
    loc = dst_ref[0] - wt * 512
    iota_n = lax.broadcasted_iota(jnp.int32, (512, 1), 0)
    ohb = (iota_n == loc)
    valid = (i * 128 + lax.broadcasted_iota(jnp.int32, (1, 128), 1)) < ub_ref[i]
    relmask = (lax.broadcasted_iota(jnp.int32, (1, R), 1) == rel)
    invcol = jnp.sum(jnp.where(relmask, invc_ref[0], 0.0), axis=1,
                     keepdims=True)
    oh = jnp.where(jnp.logical_and(ohb, valid), invcol, 0.0).astype(jnp.bfloat16)
    m = jnp.dot(xg_ref[...].astype(jnp.bfloat16), wr_ref[0],
                preferred_element_type=jnp.float32).astype(jnp.bfloat16)
    contrib = jnp.dot(oh, m, preferred_element_type=jnp.float32)

    @pl.when(first)
    def _():
        o_ref[...] = contrib[None]

    @pl.when(jnp.logical_not(first))
    def _():
        o_ref[...] += contrib[None]


def _agg1(bkt_of_blk, ub, xg, dst_s2d, wr, invc3):
    return pl.pallas_call(
        _agg1_body,
        grid_spec=pltpu.PrefetchScalarGridSpec(
            num_scalar_prefetch=2,
            grid=(NBLK,),
            in_specs=[
                pl.BlockSpec((128, D), lambda i, b, u: (i, 0)),
                pl.BlockSpec((1, 1, 128), lambda i, b, u: (i, 0, 0)),
                pl.BlockSpec((1, D, D), lambda i, b, u: (b[i] % R, 0, 0)),
                pl.BlockSpec((1, 512, R), lambda i, b, u: (b[i] // R, 0, 0)),
            ],
            out_specs=pl.BlockSpec((1, 512, D), lambda i, b, u: (b[i] // R, 0, 0)),
        ),
        out_shape=jax.ShapeDtypeStruct((WT, 512, D), jnp.float32),
    )(bkt_of_blk, ub, xg, dst_s2d, wr, invc3)


def _agg2_body(bkt_ref, ub_ref, zg_ref, dst_ref, o_ref):
    i = pl.program_id(0)
    bkt = bkt_ref[i]
    wt = bkt // R
    prevw = bkt_ref[jnp.maximum(i - 1, 0)] // R
    first = jnp.logical_or(i == 0, wt != prevw)
    loc = dst_ref[0] - wt * 512
    iota_n = lax.broadcasted_iota(jnp.int32, (512, 1), 0)
    ohb = (iota_n == loc)
    valid = (i * 128 + lax.broadcasted_iota(jnp.int32, (1, 128), 1)) < ub_ref[i]
    oh = jnp.where(jnp.logical_and(ohb, valid), 1.0, 0.0).astype(jnp.bfloat16)
    contrib = jnp.dot(oh, zg_ref[...].astype(jnp.bfloat16),
                      preferred_element_type=jnp.float32)

    @pl.when(first)
    def _():
        o_ref[...] = contrib[None]

    @pl.when(jnp.logical_not(first))
    def _():
        o_ref[...] += contrib[None]


def _agg2(bkt_of_blk, ub, zg, dst_s2d):
    return pl.pallas_call(
        _agg2_body,
        grid_spec=pltpu.PrefetchScalarGridSpec(
            num_scalar_prefetch=2,
            grid=(NBLK,),
            in_specs=[
                pl.BlockSpec((128, D), lambda i, b, u: (i, 0)),
                pl.BlockSpec((1, 1, 128), lambda i, b, u: (i, 0, 0)),
            ],
            out_specs=pl.BlockSpec((1, 512, D), lambda i, b, u: (b[i] // R, 0, 0)),
        ),
        out_shape=jax.ShapeDtypeStruct((WT, 512, D), jnp.float32),
    )(bkt_of_blk, ub, zg, dst_s2d)


def _mid_body(rg_ref, x_ref, rw_ref, rb_ref, wrel_ref, wroot_ref, z_ref, h1r_ref):
    xb = x_ref[...].astype(jnp.bfloat16)
    h1 = (rg_ref[...] + rb_ref[...]
          + jnp.dot(xb, rw_ref[...].astype(jnp.bfloat16),
                    preferred_element_type=jnp.float32))
    h1b = h1.astype(jnp.bfloat16)
    z_ref[...] = jnp.dot(h1b, wrel_ref[...].astype(jnp.bfloat16),
                         preferred_element_type=jnp.float32)
    h1r_ref[...] = jnp.dot(h1b, wroot_ref[...].astype(jnp.bfloat16),
                           preferred_element_type=jnp.float32)


def _mid(rgcn, x, root_w, rgcn_bias, gc_w_rel, gc_w_root):
    Trows = N // 10
    return pl.pallas_call(
        _mid_body,
        grid=(10,),
        in_specs=[
            pl.BlockSpec((Trows, D), lambda i: (i, 0)),
            pl.BlockSpec((Trows, D), lambda i: (i, 0)),
            pl.BlockSpec((D, D), lambda i: (0, 0)),
            pl.BlockSpec((1, D), lambda i: (0, 0)),
            pl.BlockSpec((D, D), lambda i: (0, 0)),
            pl.BlockSpec((D, D), lambda i: (0, 0)),
        ],
        out_specs=[
            pl.BlockSpec((Trows, D), lambda i: (i, 0)),
            pl.BlockSpec((Trows, D), lambda i: (i, 0)),
        ],
        out_shape=[
            jax.ShapeDtypeStruct((N, D), jnp.float32),
            jax.ShapeDtypeStruct((N, D), jnp.float32),
        ],
    )(rgcn, x, root_w, rgcn_bias.reshape(1, D), gc_w_rel, gc_w_root)


def _dec_body(x_ref, az_ref, h1r_ref, gcb_ref, w1a_ref, w1b_ref, b1_ref,
              w2_ref, b2_ref, w3_ref, b3_ref, o_ref):
    h2 = az_ref[...] + h1r_ref[...] + gcb_ref[...]
    z1 = jnp.dot(x_ref[...], w1a_ref[...], preferred_element_type=jnp.float32)
    z1 = z1 + jnp.dot(h2, w1b_ref[...], preferred_element_type=jnp.float32)
    z1 = jnp.maximum(z1 + b1_ref[...], 0.0)
    z2 = jnp.maximum(jnp.dot(z1, w2_ref[...], preferred_element_type=jnp.float32)
                     + b2_ref[...], 0.0)
    o_ref[...] = jnp.dot(z2, w3_ref[...], preferred_element_type=jnp.float32) \
        + b3_ref[...]


def _decoder(x, aggz, h1r, gc_bias, W1, b1, W2, b2, W3, b3):
    Trows = N // 10
    return pl.pallas_call(
        _dec_body,
        grid=(10,),
        in_specs=[
            pl.BlockSpec((Trows, D), lambda i: (i, 0)),
            pl.BlockSpec((Trows, D), lambda i: (i, 0)),
            pl.BlockSpec((Trows, D), lambda i: (i, 0)),
            pl.BlockSpec((1, D), lambda i: (0, 0)),
            pl.BlockSpec((D, H1), lambda i: (0, 0)),
            pl.BlockSpec((D, H1), lambda i: (0, 0)),
            pl.BlockSpec((1, H1), lambda i: (0, 0)),
            pl.BlockSpec((H1, H2), lambda i: (0, 0)),
            pl.BlockSpec((1, H2), lambda i: (0, 0)),
            pl.BlockSpec((H2, 1), lambda i: (0, 0)),
            pl.BlockSpec((1, 1), lambda i: (0, 0)),
        ],
        out_specs=pl.BlockSpec((Trows, 1), lambda i: (i, 0)),
        out_shape=jax.ShapeDtypeStruct((N, 1), jnp.float32),
    )(x, aggz, h1r, gc_bias.reshape(1, D), W1[:D], W1[D:],
      b1.reshape(1, H1), W2, b2.reshape(1, H2), W3, b3.reshape(1, 1))


def kernel(encoding, speaker, edge_index, edge_type, spk_table, bases, comp,
           root_w, rgcn_bias, gc_w_rel, gc_w_root, gc_bias,
           W1, b1, W2, b2, W3, b3):
    src = edge_index[0].astype(jnp.int32)
    dst = edge_index[1].astype(jnp.int32)
    typ = edge_type.astype(jnp.int32)
    npad = E_PAD - E
    src_p = jnp.concatenate([src, jnp.zeros((npad,), jnp.int32)])
    dst_p = jnp.concatenate([dst, jnp.full((npad,), (WT - 1) * 512, jnp.int32)])
    typ_p = jnp.concatenate([typ, jnp.zeros((npad,), jnp.int32)])

    spk_emb = _spk_emb(speaker.astype(jnp.int32).reshape(N, 1), spk_table)
    x = jnp.concatenate([encoding, spk_emb], axis=1)

    den, bk, lrk, hist = _sc_edge1(dst_p, typ_p)
    invc = _invc(den)

    hist2 = hist.reshape(NW, NBKT)
    cnt = jnp.sum(hist2, axis=0)
    caps = ((cnt + 127) // 128) * 128
    caps = jnp.where(jnp.arange(NBKT, dtype=jnp.int32) % R == 0,
                     jnp.maximum(caps, 128), caps)
    off = jnp.concatenate([jnp.zeros((1,), jnp.int32),
                           jnp.cumsum(caps)]).astype(jnp.int32)
    ub = (off[:-1] + cnt).astype(jnp.int32)
    base2 = (off[:-1][None, :]
             + (jnp.cumsum(hist2, axis=0) - hist2)).astype(jnp.int32)
    blk_starts = jnp.arange(NBLK, dtype=jnp.int32) * 128
    bkt_of_blk = (jnp.searchsorted(off[1:], blk_starts, side="right")
                  .astype(jnp.int32))
    bkt_of_blk = jnp.minimum(bkt_of_blk, NBKT - 1)
    ub_of_blk = ub[bkt_of_blk]

    pos = _pos(bk, lrk, base2)
    src_s, dst_s = _sc_permute(src_p, dst_p, pos)
    dst_s2d = dst_s.reshape(NBLK, 1, 128)

    invc3 = jnp.pad(invc[:N * R].reshape(N, R),
                    ((0, WT * 512 - N), (0, 0))).reshape(WT, 512, R)
    wr = _wr_all(comp, bases)

    xg = _sc_rowgather(x, src_s)
    agg = _agg1(bkt_of_blk, ub_of_blk, xg, dst_s2d, wr, invc3)
    rgcn = agg.reshape(WT * 512, D)[:N]

    z, h1r = _mid(rgcn, x, root_w, rgcn_bias, gc_w_rel, gc_w_root)

    zg = _sc_rowgather(z, src_s)
    aggz4 = _agg2(bkt_of_blk, ub_of_blk, zg, dst_s2d)
    aggz = aggz4.reshape(WT * 512, D)[:N]

    pred = _decoder(x, aggz, h1r, gc_bias, W1, b1, W2, b2, W3, b3)
    return pred.reshape(N)


# speedup vs baseline: 3.7180x; 1.0529x over previous
import functools

import jax
import jax.numpy as jnp
from jax import lax
from jax.experimental import pallas as pl
from jax.experimental.pallas import tpu as pltpu
from jax.experimental.pallas import tpu_sc as plsc

N = 10000
E = 160000
D = 256
R = 16
NB = 8
NC = 2
NS = 16
NW = NC * NS
E_PAD = 163840
EPT = E_PAD // NW
WT = 20 + 1
NBKT = WT * R
E_SORT = 208896
NBLK = E_SORT // 128
EST = E_SORT // NW
DENP = 160128
CHUNK = 128
H1 = 128
H2 = 64

_mesh = plsc.VectorSubcoreMesh(core_axis_name="c", subcore_axis_name="s",
                               num_cores=NC, num_subcores=NS)


def _wid():
    return lax.axis_index("c") * NS + lax.axis_index("s")


@functools.partial(
    pl.kernel,
    out_type=[
        jax.ShapeDtypeStruct((NC * DENP,), jnp.float32),
        jax.ShapeDtypeStruct((E_PAD,), jnp.int32),
        jax.ShapeDtypeStruct((E_PAD,), jnp.int32),
        jax.ShapeDtypeStruct((NW * NBKT,), jnp.int32),
    ],
    mesh=_mesh,
    scratch_types=[
        pltpu.VMEM((EPT,), jnp.int32),
        pltpu.VMEM((EPT,), jnp.int32),
        pltpu.VMEM((EPT // 128, 128), jnp.int32),
        pltpu.VMEM((EPT + 16,), jnp.int32),
        pltpu.VMEM((EPT + 16,), jnp.int32),
        pltpu.VMEM((128,), jnp.float32),
        pltpu.VMEM((NBKT + 16,), jnp.int32),
        pltpu.VMEM((5008,), jnp.float32),
        pltpu.VMEM_SHARED((DENP,), jnp.float32),
        pltpu.SemaphoreType.DMA,
    ],
)
def _sc_edge1(dst_hbm, typ_hbm, den_hbm, bk_hbm, lrk_hbm, hist_hbm,
              dst_v, typ_v, key2_v, bkt_v, lrk_v, ones_v, cnt_v, zv,
              den_sp, sem):
    cid = lax.axis_index("c")
    sid = lax.axis_index("s")
    w = cid * NS + sid
    tile_base = w * EPT
    cpA = pltpu.async_copy(dst_hbm.at[pl.ds(tile_base, EPT)], dst_v, sem)
    cpB = pltpu.async_copy(typ_hbm.at[pl.ds(tile_base, EPT)], typ_v, sem)

    def zb(j, _):
        zv[pl.ds(j * 16, 16)] = jnp.zeros((16,), jnp.float32)
        return _
    lax.fori_loop(0, 5008 // 16, zb, None)

    def zc(j, _):
        cnt_v[pl.ds(j * 16, 16)] = jnp.zeros((16,), jnp.int32)
        return _
    lax.fori_loop(0, (NBKT + 16) // 16, zc, None)

    def ob(j, _):
        ones_v[pl.ds(j * 16, 16)] = jnp.ones((16,), jnp.float32)
        return _
    lax.fori_loop(0, 128 // 16, ob, None)

    off = sid * 10008
    pltpu.sync_copy(zv, den_sp.at[pl.ds(off, 5008)])
    pltpu.sync_copy(zv.at[pl.ds(0, 5000)], den_sp.at[pl.ds(off + 5008, 5000)])
    cpA.wait()
    cpB.wait()
    plsc.subcore_barrier()

    lane0 = (lax.broadcasted_iota(jnp.int32, (16,), 0) == 0)

    def keyrow(r, _):
        def keyb(j, _):
            o = r * 128 + j * 16
            d = dst_v[pl.ds(o, 16)]
            t = typ_v[pl.ds(o, 16)]
            key2_v[r, pl.ds(j * 16, 16)] = jnp.minimum(d, N) * 16 + t
            bkt_v[pl.ds(o, 16)] = ((d >> 9) << 4) + t
            return _
        lax.fori_loop(0, 8, keyb, None)
        return _
    lax.fori_loop(0, EPT // 128, keyrow, None)
    cpC = pltpu.async_copy(bkt_v.at[pl.ds(0, EPT)],
                           bk_hbm.at[pl.ds(tile_base, EPT)], sem)
    for j in range(EPT // 128):
        pltpu.sync_copy(ones_v, den_sp.at[key2_v.at[j]], add=True)

    def rankb(e, _):
        bwin = bkt_v[pl.ds(e, 16)]
        b0 = bwin[0]
        cwin = cnt_v[pl.ds(b0, 16)]
        r0 = cwin[0]
        cnt_v[pl.ds(b0, 16)] = jnp.where(lane0, r0 + 1, cwin)
        lwin = lrk_v[pl.ds(e, 16)]
        lrk_v[pl.ds(e, 16)] = jnp.where(lane0, r0, lwin)
        return _
    lax.fori_loop(0, EPT, rankb, None)
    cpC.wait()
    pltpu.sync_copy(lrk_v.at[pl.ds(0, EPT)], lrk_hbm.at[pl.ds(tile_base, EPT)])
    pltpu.sync_copy(cnt_v.at[pl.ds(0, NBKT)], hist_hbm.at[pl.ds(w * NBKT, NBKT)])
    plsc.subcore_barrier()
    cbase = cid * DENP
    pltpu.sync_copy(den_sp.at[pl.ds(off, 5008)], zv)
    pltpu.sync_copy(zv, den_hbm.at[pl.ds(cbase + off, 5008)])
    pltpu.sync_copy(den_sp.at[pl.ds(off + 5008, 5000)], zv.at[pl.ds(0, 5000)])
    pltpu.sync_copy(zv.at[pl.ds(0, 5000)],
                    den_hbm.at[pl.ds(cbase + off + 5008, 5000)])


@functools.partial(
    pl.kernel,
    out_type=[
        jax.ShapeDtypeStruct((E_SORT,), jnp.int32),
        jax.ShapeDtypeStruct((E_SORT,), jnp.int32),
    ],
    mesh=_mesh,
    scratch_types=[
        pltpu.VMEM((EPT,), jnp.int32),
        pltpu.VMEM((EPT,), jnp.int32),
        pltpu.VMEM((EPT // 128, 128), jnp.int32),
        pltpu.SemaphoreType.DMA,
        pltpu.SemaphoreType.DMA,
    ],
)
def _sc_permute(src_hbm, dst_hbm, pos2_hbm, srcs_hbm, dsts_hbm,
                val_v, dst2_v, pos2_v, sem, sem2):
    w = _wid()
    tile_base = w * EPT
    nrow = EPT // 128
    cpA = pltpu.async_copy(src_hbm.at[pl.ds(tile_base, EPT)], val_v, sem)
    cpB = pltpu.async_copy(dst_hbm.at[pl.ds(tile_base, EPT)], dst2_v, sem)
    pltpu.sync_copy(pos2_hbm.at[pl.ds(w * nrow, nrow)], pos2_v)
    cpA.wait()
    cpB.wait()
    cps = []
    for j in range(nrow):
        cps.append(pltpu.async_copy(val_v.at[pl.ds(j * 128, 128)],
                                    srcs_hbm.at[pos2_v.at[j]], sem))
        cps.append(pltpu.async_copy(dst2_v.at[pl.ds(j * 128, 128)],
                                    dsts_hbm.at[pos2_v.at[j]], sem2))
        while len(cps) > 6:
            cps.pop(0).wait()
    for cp in cps:
        cp.wait()


@functools.partial(
    pl.kernel,
    out_type=jax.ShapeDtypeStruct((E_SORT, D), jnp.float32),
    mesh=_mesh,
    scratch_types=[
        pltpu.VMEM((EST,), jnp.int32),
        pltpu.VMEM((192, D), jnp.float32),
        pltpu.VMEM((192, D), jnp.float32),
        pltpu.SemaphoreType.DMA,
        pltpu.SemaphoreType.DMA,
        pltpu.SemaphoreType.DMA,
        pltpu.SemaphoreType.DMA,
    ],
)
def _sc_rowgather(tab_hbm, idx_hbm, out_hbm, idx_v, buf0, buf1,
                  gs0, gs1, ws0, ws1):
    w = _wid()
    tile_base = w * EST
    GC = 192
    nch = EST // GC
    pltpu.sync_copy(idx_hbm.at[pl.ds(tile_base, EST)], idx_v)

    def clampb(j, _):
        o = j * 16
        v = idx_v[pl.ds(o, 16)]
        idx_v[pl.ds(o, 16)] = jnp.clip(v, 0, N - 1)
        return _
    lax.fori_loop(0, EST // 16, clampb, None)

    bufs = (buf0, buf1)
    gsems = (gs0, gs1)
    wsems = (ws0, ws1)
    gd = [None, None]
    wd = [None, None]
    for i in range(nch):
        b = i % 2
        if wd[b] is not None:
            wd[b].wait()
        gd[b] = pltpu.async_copy(
            tab_hbm.at[idx_v.at[pl.ds(i * GC, GC)]], bufs[b], gsems[b])
        if i >= 1:
            pb = 1 - b
            gd[pb].wait()
            wd[pb] = pltpu.async_copy(
                bufs[pb], out_hbm.at[pl.ds(tile_base + (i - 1) * GC, GC)],
                wsems[pb])
    lb = (nch - 1) % 2
    gd[lb].wait()
    wd[lb] = pltpu.async_copy(
        bufs[lb], out_hbm.at[pl.ds(tile_base + (nch - 1) * GC, GC)], wsems[lb])
    wd[0].wait()
    wd[1].wait()


def _spk_body(spk_ref, tab_ref, o_ref):
    s = spk_ref[...]
    o_ref[...] = jnp.where(s == 0, tab_ref[0:1, :], tab_ref[1:2, :])


def _spk_emb(speaker2d, spk_table):
    return pl.pallas_call(
        _spk_body,
        grid=(10,),
        in_specs=[
            pl.BlockSpec((N // 10, 1), lambda i: (i, 0)),
            pl.BlockSpec((2, 32), lambda i: (0, 0)),
        ],
        out_specs=pl.BlockSpec((N // 10, 32), lambda i: (i, 0)),
        out_shape=jax.ShapeDtypeStruct((N, 32), jnp.float32),
    )(speaker2d, spk_table)


def _invc_body(d_ref, o_ref):
    s = d_ref[0] + d_ref[1]
    o_ref[...] = 1.0 / jnp.maximum(s, 1.0)


def _invc(den):
    return pl.pallas_call(
        _invc_body,
        out_shape=jax.ShapeDtypeStruct((DENP // 128, 128), jnp.float32),
    )(den.reshape(2, DENP // 128, 128)).reshape(DENP)


def _wr_body(comp_ref, bases_ref, o_ref):
    o_ref[...] = jnp.dot(comp_ref[...], bases_ref[...],
                         preferred_element_type=jnp.float32)


def _wr_all(comp, bases):
    out = pl.pallas_call(
        _wr_body,
        out_shape=jax.ShapeDtypeStruct((R, D * D), jnp.float32),
    )(comp, bases.reshape(NB, D * D))
    return out.reshape(R, D, D).astype(jnp.bfloat16)


def _pos_body(bk_ref, lrk_ref, b2_ref, o_ref):
    oh = (bk_ref[...] == lax.broadcasted_iota(jnp.int32, (1, NBKT), 1))
    row = b2_ref[0]
    o_ref[...] = jnp.sum(jnp.where(oh, row, 0), axis=1,
                         keepdims=True) + lrk_ref[...]


def _pos(bk, lrk, base2):
    return pl.pallas_call(
        _pos_body,
        grid=(NW,),
        in_specs=[
            pl.BlockSpec((EPT, 1), lambda i: (i, 0)),
            pl.BlockSpec((EPT, 1), lambda i: (i, 0)),
            pl.BlockSpec((1, 1, NBKT), lambda i: (i, 0, 0)),
        ],
        out_specs=pl.BlockSpec((EPT, 1), lambda i: (i, 0)),
        out_shape=jax.ShapeDtypeStruct((E_PAD, 1), jnp.int32),
    )(bk.reshape(E_PAD, 1), lrk.reshape(E_PAD, 1),
      base2.reshape(NW, 1, NBKT)).reshape(E_PAD)


def _agg1_body(bkt_ref, ub_ref, xg_ref, dst_ref, wr_ref, invc_ref, o_ref):
    i = pl.program_id(0)
    bkt = bkt_ref[i]
    rel = bkt % R
    wt = bkt // R
    prevw = bkt_ref[jnp.maximum(i - 1, 0)] // R
    first = jnp.logical_or(i == 0, wt != prevw)
    loc = dst_ref[0] - wt * 512
    iota_n = lax.broadcasted_iota(jnp.int32, (512, 1), 0)
    ohb = (iota_n == loc)
    valid = (i * 128 + lax.broadcasted_iota(jnp.int32, (1, 128), 1)) < ub_ref[i]
    relmask = (lax.broadcasted_iota(jnp.int32, (1, R), 1) == rel)
    invcol = jnp.sum(jnp.where(relmask, invc_ref[0], 0.0), axis=1,
                     keepdims=True)
    oh = jnp.where(jnp.logical_and(ohb, valid), invcol, 0.0).astype(jnp.bfloat16)
    m = jnp.dot(xg_ref[...].astype(jnp.bfloat16), wr_ref[0],
                preferred_element_type=jnp.float32).astype(jnp.bfloat16)
    contrib = jnp.dot(oh, m, preferred_element_type=jnp.float32)

    @pl.when(first)
    def _():
        o_ref[...] = contrib[None]

    @pl.when(jnp.logical_not(first))
    def _():
        o_ref[...] += contrib[None]


def _agg1(bkt_of_blk, ub, xg, dst_s2d, wr, invc3):
    return pl.pallas_call(
        _agg1_body,
        grid_spec=pltpu.PrefetchScalarGridSpec(
            num_scalar_prefetch=2,
            grid=(NBLK,),
            in_specs=[
                pl.BlockSpec((128, D), lambda i, b, u: (i, 0)),
                pl.BlockSpec((1, 1, 128), lambda i, b, u: (i, 0, 0)),
                pl.BlockSpec((1, D, D), lambda i, b, u: (b[i] % R, 0, 0)),
                pl.BlockSpec((1, 512, R), lambda i, b, u: (b[i] // R, 0, 0)),
            ],
            out_specs=pl.BlockSpec((1, 512, D), lambda i, b, u: (b[i] // R, 0, 0)),
        ),
        out_shape=jax.ShapeDtypeStruct((WT, 512, D), jnp.float32),
    )(bkt_of_blk, ub, xg, dst_s2d, wr, invc3)


def _agg2_body(bkt_ref, ub_ref, zg_ref, dst_ref, o_ref):
    i = pl.program_id(0)
    bkt = bkt_ref[i]
    wt = bkt // R
    prevw = bkt_ref[jnp.maximum(i - 1, 0)] // R
    first = jnp.logical_or(i == 0, wt != prevw)
    loc = dst_ref[0] - wt * 512
    iota_n = lax.broadcasted_iota(jnp.int32, (512, 1), 0)
    ohb = (iota_n == loc)
    valid = (i * 128 + lax.broadcasted_iota(jnp.int32, (1, 128), 1)) < ub_ref[i]
    oh = jnp.where(jnp.logical_and(ohb, valid), 1.0, 0.0).astype(jnp.bfloat16)
    contrib = jnp.dot(oh, zg_ref[...].astype(jnp.bfloat16),
                      preferred_element_type=jnp.float32)

    @pl.when(first)
    def _():
        o_ref[...] = contrib[None]

    @pl.when(jnp.logical_not(first))
    def _():
        o_ref[...] += contrib[None]


def _agg2(bkt_of_blk, ub, zg, dst_s2d):
    return pl.pallas_call(
        _agg2_body,
        grid_spec=pltpu.PrefetchScalarGridSpec(
            num_scalar_prefetch=2,
            grid=(NBLK,),
            in_specs=[
                pl.BlockSpec((128, D), lambda i, b, u: (i, 0)),
                pl.BlockSpec((1, 1, 128), lambda i, b, u: (i, 0, 0)),
            ],
            out_specs=pl.BlockSpec((1, 512, D), lambda i, b, u: (b[i] // R, 0, 0)),
        ),
        out_shape=jax.ShapeDtypeStruct((WT, 512, D), jnp.float32),
    )(bkt_of_blk, ub, zg, dst_s2d)


def _mid_body(rg_ref, x_ref, rw_ref, rb_ref, wrel_ref, wroot_ref, z_ref, h1r_ref):
    xb = x_ref[...].astype(jnp.bfloat16)
    h1 = (rg_ref[...] + rb_ref[...]
          + jnp.dot(xb, rw_ref[...].astype(jnp.bfloat16),
                    preferred_element_type=jnp.float32))
    h1b = h1.astype(jnp.bfloat16)
    z_ref[...] = jnp.dot(h1b, wrel_ref[...].astype(jnp.bfloat16),
                         preferred_element_type=jnp.float32)
    h1r_ref[...] = jnp.dot(h1b, wroot_ref[...].astype(jnp.bfloat16),
                           preferred_element_type=jnp.float32)


def _mid(rgcn, x, root_w, rgcn_bias, gc_w_rel, gc_w_root):
    Trows = N // 10
    return pl.pallas_call(
        _mid_body,
        grid=(10,),
        in_specs=[
            pl.BlockSpec((Trows, D), lambda i: (i, 0)),
            pl.BlockSpec((Trows, D), lambda i: (i, 0)),
            pl.BlockSpec((D, D), lambda i: (0, 0)),
            pl.BlockSpec((1, D), lambda i: (0, 0)),
            pl.BlockSpec((D, D), lambda i: (0, 0)),
            pl.BlockSpec((D, D), lambda i: (0, 0)),
        ],
        out_specs=[
            pl.BlockSpec((Trows, D), lambda i: (i, 0)),
            pl.BlockSpec((Trows, D), lambda i: (i, 0)),
        ],
        out_shape=[
            jax.ShapeDtypeStruct((N, D), jnp.float32),
            jax.ShapeDtypeStruct((N, D), jnp.float32),
        ],
    )(rgcn, x, root_w, rgcn_bias.reshape(1, D), gc_w_rel, gc_w_root)


def _dec_body(x_ref, az_ref, h1r_ref, gcb_ref, w1a_ref, w1b_ref, b1_ref,
              w2_ref, b2_ref, w3_ref, b3_ref, o_ref):
    h2 = az_ref[...] + h1r_ref[...] + gcb_ref[...]
    z1 = jnp.dot(x_ref[...], w1a_ref[...], preferred_element_type=jnp.float32)
    z1 = z1 + jnp.dot(h2, w1b_ref[...], preferred_element_type=jnp.float32)
    z1 = jnp.maximum(z1 + b1_ref[...], 0.0)
    z2 = jnp.maximum(jnp.dot(z1, w2_ref[...], preferred_element_type=jnp.float32)
                     + b2_ref[...], 0.0)
    o_ref[...] = jnp.dot(z2, w3_ref[...], preferred_element_type=jnp.float32) \
        + b3_ref[...]


def _decoder(x, aggz, h1r, gc_bias, W1, b1, W2, b2, W3, b3):
    Trows = N // 10
    return pl.pallas_call(
        _dec_body,
        grid=(10,),
        in_specs=[
            pl.BlockSpec((Trows, D), lambda i: (i, 0)),
            pl.BlockSpec((Trows, D), lambda i: (i, 0)),
            pl.BlockSpec((Trows, D), lambda i: (i, 0)),
            pl.BlockSpec((1, D), lambda i: (0, 0)),
            pl.BlockSpec((D, H1), lambda i: (0, 0)),
            pl.BlockSpec((D, H1), lambda i: (0, 0)),
            pl.BlockSpec((1, H1), lambda i: (0, 0)),
            pl.BlockSpec((H1, H2), lambda i: (0, 0)),
            pl.BlockSpec((1, H2), lambda i: (0, 0)),
            pl.BlockSpec((H2, 1), lambda i: (0, 0)),
            pl.BlockSpec((1, 1), lambda i: (0, 0)),
        ],
        out_specs=pl.BlockSpec((Trows, 1), lambda i: (i, 0)),
        out_shape=jax.ShapeDtypeStruct((N, 1), jnp.float32),
    )(x, aggz, h1r, gc_bias.reshape(1, D), W1[:D], W1[D:],
      b1.reshape(1, H1), W2, b2.reshape(1, H2), W3, b3.reshape(1, 1))


def kernel(encoding, speaker, edge_index, edge_type, spk_table, bases, comp,
           root_w, rgcn_bias, gc_w_rel, gc_w_root, gc_bias,
           W1, b1, W2, b2, W3, b3):
    src = edge_index[0].astype(jnp.int32)
    dst = edge_index[1].astype(jnp.int32)
    typ = edge_type.astype(jnp.int32)
    npad = E_PAD - E
    src_p = jnp.concatenate([src, jnp.zeros((npad,), jnp.int32)])
    dst_p = jnp.concatenate([dst, jnp.full((npad,), (WT - 1) * 512, jnp.int32)])
    typ_p = jnp.concatenate([typ, jnp.zeros((npad,), jnp.int32)])

    spk_emb = _spk_emb(speaker.astype(jnp.int32).reshape(N, 1), spk_table)
    x = jnp.concatenate([encoding, spk_emb], axis=1)

    den, bk, lrk, hist = _sc_edge1(dst_p, typ_p)
    invc = _invc(den)

    hist2 = hist.reshape(NW, NBKT)
    cnt = jnp.sum(hist2, axis=0)
    caps = ((cnt + 127) // 128) * 128
    caps = jnp.where(jnp.arange(NBKT, dtype=jnp.int32) % R == 0,
                     jnp.maximum(caps, 128), caps)
    off = jnp.concatenate([jnp.zeros((1,), jnp.int32),
                           jnp.cumsum(caps)]).astype(jnp.int32)
    ub = (off[:-1] + cnt).astype(jnp.int32)
    base2 = (off[:-1][None, :]
             + (jnp.cumsum(hist2, axis=0) - hist2)).astype(jnp.int32)
    blk_starts = jnp.arange(NBLK, dtype=jnp.int32) * 128
    bkt_of_blk = (jnp.searchsorted(off[1:], blk_starts, side="right")
                  .astype(jnp.int32))
    bkt_of_blk = jnp.minimum(bkt_of_blk, NBKT - 1)
    ub_of_blk = ub[bkt_of_blk]

    pos = _pos(bk, lrk, base2)
    src_s, dst_s = _sc_permute(src_p, dst_p, pos.reshape(E_PAD // 128, 128))
    dst_s2d = dst_s.reshape(NBLK, 1, 128)

    invc3 = jnp.pad(invc[:N * R].reshape(N, R),
                    ((0, WT * 512 - N), (0, 0))).reshape(WT, 512, R)
    wr = _wr_all(comp, bases)

    xg = _sc_rowgather(x, src_s)
    agg = _agg1(bkt_of_blk, ub_of_blk, xg, dst_s2d, wr, invc3)
    rgcn = agg.reshape(WT * 512, D)[:N]

    z, h1r = _mid(rgcn, x, root_w, rgcn_bias, gc_w_rel, gc_w_root)

    zg = _sc_rowgather(z, src_s)
    aggz4 = _agg2(bkt_of_blk, ub_of_blk, zg, dst_s2d)
    aggz = aggz4.reshape(WT * 512, D)[:N]

    pred = _decoder(x, aggz, h1r, gc_bias, W1, b1, W2, b2, W3, b3)
    return pred.reshape(N)


# speedup vs baseline: 3.9577x; 1.0645x over previous
import functools

import jax
import jax.numpy as jnp
from jax import lax
from jax.experimental import pallas as pl
from jax.experimental.pallas import tpu as pltpu
from jax.experimental.pallas import tpu_sc as plsc

N = 10000
E = 160000
D = 256
R = 16
NB = 8
NC = 2
NS = 16
NW = NC * NS
E_PAD = 163840
EPT = E_PAD // NW
WT = 20 + 1
NBKT = WT * R
E_SORT = 208896
NBLK = E_SORT // 128
EST = E_SORT // NW
DENP = 160128
CHUNK = 128
H1 = 128
H2 = 64

_mesh = plsc.VectorSubcoreMesh(core_axis_name="c", subcore_axis_name="s",
                               num_cores=NC, num_subcores=NS)


def _wid():
    return lax.axis_index("c") * NS + lax.axis_index("s")


@functools.partial(
    pl.kernel,
    out_type=[
        jax.ShapeDtypeStruct((NC * DENP,), jnp.float32),
        jax.ShapeDtypeStruct((E_PAD,), jnp.int32),
        jax.ShapeDtypeStruct((E_PAD,), jnp.int32),
        jax.ShapeDtypeStruct((NW * NBKT,), jnp.int32),
    ],
    mesh=_mesh,
    scratch_types=[
        pltpu.VMEM((EPT,), jnp.int32),
        pltpu.VMEM((EPT,), jnp.int32),
        pltpu.VMEM((EPT // 128, 128), jnp.int32),
        pltpu.VMEM((EPT + 16,), jnp.int32),
        pltpu.VMEM((EPT + 16,), jnp.int32),
        pltpu.VMEM((128,), jnp.float32),
        pltpu.VMEM((NBKT + 16,), jnp.int32),
        pltpu.VMEM((5008,), jnp.float32),
        pltpu.VMEM_SHARED((DENP,), jnp.float32),
        pltpu.SemaphoreType.DMA,
    ],
)
def _sc_edge1(dst_hbm, typ_hbm, den_hbm, bk_hbm, lrk_hbm, hist_hbm,
              dst_v, typ_v, key2_v, bkt_v, lrk_v, ones_v, cnt_v, zv,
              den_sp, sem):
    cid = lax.axis_index("c")
    sid = lax.axis_index("s")
    w = cid * NS + sid
    tile_base = w * EPT
    cpA = pltpu.async_copy(dst_hbm.at[pl.ds(tile_base, EPT)], dst_v, sem)
    cpB = pltpu.async_copy(typ_hbm.at[pl.ds(tile_base, EPT)], typ_v, sem)

    def zb(j, _):
        zv[pl.ds(j * 16, 16)] = jnp.zeros((16,), jnp.float32)
        return _
    lax.fori_loop(0, 5008 // 16, zb, None)

    def zc(j, _):
        cnt_v[pl.ds(j * 16, 16)] = jnp.zeros((16,), jnp.int32)
        return _
    lax.fori_loop(0, (NBKT + 16) // 16, zc, None)

    def ob(j, _):
        ones_v[pl.ds(j * 16, 16)] = jnp.ones((16,), jnp.float32)
        return _
    lax.fori_loop(0, 128 // 16, ob, None)

    off = sid * 10008
    pltpu.sync_copy(zv, den_sp.at[pl.ds(off, 5008)])
    pltpu.sync_copy(zv.at[pl.ds(0, 5000)], den_sp.at[pl.ds(off + 5008, 5000)])
    cpA.wait()
    cpB.wait()
    plsc.subcore_barrier()

    lane0 = (lax.broadcasted_iota(jnp.int32, (16,), 0) == 0)

    def keyrow(r, _):
        def keyb(j, _):
            o = r * 128 + j * 16
            d = dst_v[pl.ds(o, 16)]
            t = typ_v[pl.ds(o, 16)]
            key2_v[r, pl.ds(j * 16, 16)] = jnp.minimum(d, N) * 16 + t
            bkt_v[pl.ds(o, 16)] = ((d >> 9) << 4) + t
            return _
        lax.fori_loop(0, 8, keyb, None)
        return _
    lax.fori_loop(0, EPT // 128, keyrow, None)
    cpC = pltpu.async_copy(bkt_v.at[pl.ds(0, EPT)],
                           bk_hbm.at[pl.ds(tile_base, EPT)], sem)
    for j in range(EPT // 128):
        pltpu.sync_copy(ones_v, den_sp.at[key2_v.at[j]], add=True)

    def rankb(e, _):
        bwin = bkt_v[pl.ds(e, 16)]
        b0 = bwin[0]
        cwin = cnt_v[pl.ds(b0, 16)]
        r0 = cwin[0]
        cnt_v[pl.ds(b0, 16)] = jnp.where(lane0, r0 + 1, cwin)
        lwin = lrk_v[pl.ds(e, 16)]
        lrk_v[pl.ds(e, 16)] = jnp.where(lane0, r0, lwin)
        return _
    lax.fori_loop(0, EPT, rankb, None)
    cpC.wait()
    pltpu.sync_copy(lrk_v.at[pl.ds(0, EPT)], lrk_hbm.at[pl.ds(tile_base, EPT)])
    pltpu.sync_copy(cnt_v.at[pl.ds(0, NBKT)], hist_hbm.at[pl.ds(w * NBKT, NBKT)])
    plsc.subcore_barrier()
    cbase = cid * DENP
    pltpu.sync_copy(den_sp.at[pl.ds(off, 5008)], zv)
    pltpu.sync_copy(zv, den_hbm.at[pl.ds(cbase + off, 5008)])
    pltpu.sync_copy(den_sp.at[pl.ds(off + 5008, 5000)], zv.at[pl.ds(0, 5000)])
    pltpu.sync_copy(zv.at[pl.ds(0, 5000)],
                    den_hbm.at[pl.ds(cbase + off + 5008, 5000)])


@functools.partial(
    pl.kernel,
    out_type=jax.ShapeDtypeStruct((E_SORT,), jnp.int32),
    mesh=_mesh,
    scratch_types=[
        pltpu.VMEM((EPT,), jnp.int32),
        pltpu.VMEM((EPT,), jnp.int32),
        pltpu.VMEM((EPT // 128, 128), jnp.int32),
        pltpu.SemaphoreType.DMA,
        pltpu.SemaphoreType.DMA,
    ],
)
def _sc_permute(src_hbm, dst_hbm, pos2_hbm, ed_hbm,
                val_v, dst2_v, pos2_v, sem, sem2):
    w = _wid()
    tile_base = w * EPT
    nrow = EPT // 128
    cpA = pltpu.async_copy(src_hbm.at[pl.ds(tile_base, EPT)], val_v, sem)
    cpB = pltpu.async_copy(dst_hbm.at[pl.ds(tile_base, EPT)], dst2_v, sem)
    pltpu.sync_copy(pos2_hbm.at[pl.ds(w * nrow, nrow)], pos2_v)
    cpA.wait()
    cpB.wait()

    def packb(j, _):
        o = j * 16
        val_v[pl.ds(o, 16)] = (val_v[pl.ds(o, 16)] + 1) \
            + (dst2_v[pl.ds(o, 16)] << 14)
        return _
    lax.fori_loop(0, EPT // 16, packb, None)
    cps = []
    for j in range(nrow):
        cps.append(pltpu.async_copy(val_v.at[pl.ds(j * 128, 128)],
                                    ed_hbm.at[pos2_v.at[j]], sem))
        while len(cps) > 6:
            cps.pop(0).wait()
    for cp in cps:
        cp.wait()


@functools.partial(
    pl.kernel,
    out_type=jax.ShapeDtypeStruct((E_SORT, D), jnp.float32),
    mesh=_mesh,
    scratch_types=[
        pltpu.VMEM((EST,), jnp.int32),
        pltpu.VMEM((192, D), jnp.float32),
        pltpu.VMEM((192, D), jnp.float32),
        pltpu.SemaphoreType.DMA,
        pltpu.SemaphoreType.DMA,
        pltpu.SemaphoreType.DMA,
        pltpu.SemaphoreType.DMA,
    ],
)
def _sc_rowgather(tab_hbm, idx_hbm, out_hbm, idx_v, buf0, buf1,
                  gs0, gs1, ws0, ws1):
    w = _wid()
    tile_base = w * EST
    GC = 192
    nch = EST // GC
    pltpu.sync_copy(idx_hbm.at[pl.ds(tile_base, EST)], idx_v)

    lanes = lax.broadcasted_iota(jnp.int32, (16,), 0)

    def clampb(j, _):
        o = j * 16
        pk = idx_v[pl.ds(o, 16)]
        v = jnp.clip((pk & 0x3FFF) - 1, 0, N - 1)
        spread = (o + lanes + tile_base) & 0x1FFF
        idx_v[pl.ds(o, 16)] = jnp.where(pk == 0, spread, v)
        return _
    lax.fori_loop(0, EST // 16, clampb, None)

    bufs = (buf0, buf1)
    gsems = (gs0, gs1)
    wsems = (ws0, ws1)
    gd = [None, None]
    wd = [None, None]
    for i in range(nch):
        b = i % 2
        if wd[b] is not None:
            wd[b].wait()
        gd[b] = pltpu.async_copy(
            tab_hbm.at[idx_v.at[pl.ds(i * GC, GC)]], bufs[b], gsems[b])
        if i >= 1:
            pb = 1 - b
            gd[pb].wait()
            wd[pb] = pltpu.async_copy(
                bufs[pb], out_hbm.at[pl.ds(tile_base + (i - 1) * GC, GC)],
                wsems[pb])
    lb = (nch - 1) % 2
    gd[lb].wait()
    wd[lb] = pltpu.async_copy(
        bufs[lb], out_hbm.at[pl.ds(tile_base + (nch - 1) * GC, GC)], wsems[lb])
    wd[0].wait()
    wd[1].wait()


def _spk_body(spk_ref, tab_ref, o_ref):
    s = spk_ref[...]
    o_ref[...] = jnp.where(s == 0, tab_ref[0:1, :], tab_ref[1:2, :])


def _spk_emb(speaker2d, spk_table):
    return pl.pallas_call(
        _spk_body,
        grid=(10,),
        in_specs=[
            pl.BlockSpec((N // 10, 1), lambda i: (i, 0)),
            pl.BlockSpec((2, 32), lambda i: (0, 0)),
        ],
        out_specs=pl.BlockSpec((N // 10, 32), lambda i: (i, 0)),
        out_shape=jax.ShapeDtypeStruct((N, 32), jnp.float32),
    )(speaker2d, spk_table)


def _invc_body(d_ref, o_ref):
    s = d_ref[0] + d_ref[1]
    o_ref[...] = 1.0 / jnp.maximum(s, 1.0)


def _invc(den):
    return pl.pallas_call(
        _invc_body,
        out_shape=jax.ShapeDtypeStruct((DENP // 128, 128), jnp.float32),
    )(den.reshape(2, DENP // 128, 128)).reshape(DENP)


def _wr_body(comp_ref, bases_ref, o_ref):
    o_ref[...] = jnp.dot(comp_ref[...], bases_ref[...],
                         preferred_element_type=jnp.float32)


def _wr_all(comp, bases):
    out = pl.pallas_call(
        _wr_body,
        out_shape=jax.ShapeDtypeStruct((R, D * D), jnp.float32),
    )(comp, bases.reshape(NB, D * D))
    return out.reshape(R, D, D).astype(jnp.bfloat16)


def _pos_body(bk_ref, lrk_ref, b2_ref, o_ref):
    oh = (bk_ref[...] == lax.broadcasted_iota(jnp.int32, (1, NBKT), 1))
    row = b2_ref[0]
    o_ref[...] = jnp.sum(jnp.where(oh, row, 0), axis=1,
                         keepdims=True) + lrk_ref[...]


def _pos(bk, lrk, base2):
    return pl.pallas_call(
        _pos_body,
        grid=(NW,),
        in_specs=[
            pl.BlockSpec((EPT, 1), lambda i: (i, 0)),
            pl.BlockSpec((EPT, 1), lambda i: (i, 0)),
            pl.BlockSpec((1, 1, NBKT), lambda i: (i, 0, 0)),
        ],
        out_specs=pl.BlockSpec((EPT, 1), lambda i: (i, 0)),
        out_shape=jax.ShapeDtypeStruct((E_PAD, 1), jnp.int32),
    )(bk.reshape(E_PAD, 1), lrk.reshape(E_PAD, 1),
      base2.reshape(NW, 1, NBKT)).reshape(E_PAD)


def _agg1_body(bkt_ref, ub_ref, xg_ref, dst_ref, wr_ref, invc_ref, o_ref):
    i = pl.program_id(0)
    bkt = bkt_ref[i]
    rel = bkt % R
    wt = bkt // R
    prevw = bkt_ref[jnp.maximum(i - 1, 0)] // R
    first = jnp.logical_or(i == 0, wt != prevw)
    loc = (dst_ref[0] >> 14) - wt * 512
    iota_n = lax.broadcasted_iota(jnp.int32, (512, 1), 0)
    ohb = (iota_n == loc)
    valid = (i * 128 + lax.broadcasted_iota(jnp.int32, (1, 128), 1)) < ub_ref[i]
    relmask = (lax.broadcasted_iota(jnp.int32, (1, R), 1) == rel)
    invcol = jnp.sum(jnp.where(relmask, invc_ref[0], 0.0), axis=1,
                     keepdims=True)
    oh = jnp.where(jnp.logical_and(ohb, valid), invcol, 0.0).astype(jnp.bfloat16)
    m = jnp.dot(xg_ref[...].astype(jnp.bfloat16), wr_ref[0],
                preferred_element_type=jnp.float32).astype(jnp.bfloat16)
    contrib = jnp.dot(oh, m, preferred_element_type=jnp.float32)

    @pl.when(first)
    def _():
        o_ref[...] = contrib[None]

    @pl.when(jnp.logical_not(first))
    def _():
        o_ref[...] += contrib[None]


def _agg1(bkt_of_blk, ub, xg, dst_s2d, wr, invc3):
    return pl.pallas_call(
        _agg1_body,
        grid_spec=pltpu.PrefetchScalarGridSpec(
            num_scalar_prefetch=2,
            grid=(NBLK,),
            in_specs=[
                pl.BlockSpec((128, D), lambda i, b, u: (i, 0)),
                pl.BlockSpec((1, 1, 128), lambda i, b, u: (i, 0, 0)),
                pl.BlockSpec((1, D, D), lambda i, b, u: (b[i] % R, 0, 0)),
                pl.BlockSpec((1, 512, R), lambda i, b, u: (b[i] // R, 0, 0)),
            ],
            out_specs=pl.BlockSpec((1, 512, D), lambda i, b, u: (b[i] // R, 0, 0)),
        ),
        out_shape=jax.ShapeDtypeStruct((WT, 512, D), jnp.float32),
    )(bkt_of_blk, ub, xg, dst_s2d, wr, invc3)


def _agg2_body(bkt_ref, ub_ref, zg_ref, dst_ref, o_ref):
    i = pl.program_id(0)
    bkt = bkt_ref[i]
    wt = bkt // R
    prevw = bkt_ref[jnp.maximum(i - 1, 0)] // R
    first = jnp.logical_or(i == 0, wt != prevw)
    loc = (dst_ref[0] >> 14) - wt * 512
    iota_n = lax.broadcasted_iota(jnp.int32, (512, 1), 0)
    ohb = (iota_n == loc)
    valid = (i * 128 + lax.broadcasted_iota(jnp.int32, (1, 128), 1)) < ub_ref[i]
    oh = jnp.where(jnp.logical_and(ohb, valid), 1.0, 0.0).astype(jnp.bfloat16)
    contrib = jnp.dot(oh, zg_ref[...].astype(jnp.bfloat16),
                      preferred_element_type=jnp.float32)

    @pl.when(first)
    def _():
        o_ref[...] = contrib[None]

    @pl.when(jnp.logical_not(first))
    def _():
        o_ref[...] += contrib[None]


def _agg2(bkt_of_blk, ub, zg, dst_s2d):
    return pl.pallas_call(
        _agg2_body,
        grid_spec=pltpu.PrefetchScalarGridSpec(
            num_scalar_prefetch=2,
            grid=(NBLK,),
            in_specs=[
                pl.BlockSpec((128, D), lambda i, b, u: (i, 0)),
                pl.BlockSpec((1, 1, 128), lambda i, b, u: (i, 0, 0)),
            ],
            out_specs=pl.BlockSpec((1, 512, D), lambda i, b, u: (b[i] // R, 0, 0)),
        ),
        out_shape=jax.ShapeDtypeStruct((WT, 512, D), jnp.float32),
    )(bkt_of_blk, ub, zg, dst_s2d)


def _mid_body(rg_ref, x_ref, rw_ref, rb_ref, wrel_ref, wroot_ref, z_ref, h1r_ref):
    xb = x_ref[...].astype(jnp.bfloat16)
    h1 = (rg_ref[...] + rb_ref[...]
          + jnp.dot(xb, rw_ref[...].astype(jnp.bfloat16),
                    preferred_element_type=jnp.float32))
    h1b = h1.astype(jnp.bfloat16)
    z_ref[...] = jnp.dot(h1b, wrel_ref[...].astype(jnp.bfloat16),
                         preferred_element_type=jnp.float32)
    h1r_ref[...] = jnp.dot(h1b, wroot_ref[...].astype(jnp.bfloat16),
                           preferred_element_type=jnp.float32)


def _mid(rgcn, x, root_w, rgcn_bias, gc_w_rel, gc_w_root):
    Trows = N // 10
    return pl.pallas_call(
        _mid_body,
        grid=(10,),
        in_specs=[
            pl.BlockSpec((Trows, D), lambda i: (i, 0)),
            pl.BlockSpec((Trows, D), lambda i: (i, 0)),
            pl.BlockSpec((D, D), lambda i: (0, 0)),
            pl.BlockSpec((1, D), lambda i: (0, 0)),
            pl.BlockSpec((D, D), lambda i: (0, 0)),
            pl.BlockSpec((D, D), lambda i: (0, 0)),
        ],
        out_specs=[
            pl.BlockSpec((Trows, D), lambda i: (i, 0)),
            pl.BlockSpec((Trows, D), lambda i: (i, 0)),
        ],
        out_shape=[
            jax.ShapeDtypeStruct((N, D), jnp.float32),
            jax.ShapeDtypeStruct((N, D), jnp.float32),
        ],
    )(rgcn, x, root_w, rgcn_bias.reshape(1, D), gc_w_rel, gc_w_root)


def _dec_body(x_ref, az_ref, h1r_ref, gcb_ref, w1a_ref, w1b_ref, b1_ref,
              w2_ref, b2_ref, w3_ref, b3_ref, o_ref):
    h2 = az_ref[...] + h1r_ref[...] + gcb_ref[...]
    z1 = jnp.dot(x_ref[...], w1a_ref[...], preferred_element_type=jnp.float32)
    z1 = z1 + jnp.dot(h2, w1b_ref[...], preferred_element_type=jnp.float32)
    z1 = jnp.maximum(z1 + b1_ref[...], 0.0)
    z2 = jnp.maximum(jnp.dot(z1, w2_ref[...], preferred_element_type=jnp.float32)
                     + b2_ref[...], 0.0)
    o_ref[...] = jnp.dot(z2, w3_ref[...], preferred_element_type=jnp.float32) \
        + b3_ref[...]


def _decoder(x, aggz, h1r, gc_bias, W1, b1, W2, b2, W3, b3):
    Trows = N // 10
    return pl.pallas_call(
        _dec_body,
        grid=(10,),
        in_specs=[
            pl.BlockSpec((Trows, D), lambda i: (i, 0)),
            pl.BlockSpec((Trows, D), lambda i: (i, 0)),
            pl.BlockSpec((Trows, D), lambda i: (i, 0)),
            pl.BlockSpec((1, D), lambda i: (0, 0)),
            pl.BlockSpec((D, H1), lambda i: (0, 0)),
            pl.BlockSpec((D, H1), lambda i: (0, 0)),
            pl.BlockSpec((1, H1), lambda i: (0, 0)),
            pl.BlockSpec((H1, H2), lambda i: (0, 0)),
            pl.BlockSpec((1, H2), lambda i: (0, 0)),
            pl.BlockSpec((H2, 1), lambda i: (0, 0)),
            pl.BlockSpec((1, 1), lambda i: (0, 0)),
        ],
        out_specs=pl.BlockSpec((Trows, 1), lambda i: (i, 0)),
        out_shape=jax.ShapeDtypeStruct((N, 1), jnp.float32),
    )(x, aggz, h1r, gc_bias.reshape(1, D), W1[:D], W1[D:],
      b1.reshape(1, H1), W2, b2.reshape(1, H2), W3, b3.reshape(1, 1))


def kernel(encoding, speaker, edge_index, edge_type, spk_table, bases, comp,
           root_w, rgcn_bias, gc_w_rel, gc_w_root, gc_bias,
           W1, b1, W2, b2, W3, b3):
    src = edge_index[0].astype(jnp.int32)
    dst = edge_index[1].astype(jnp.int32)
    typ = edge_type.astype(jnp.int32)
    npad = E_PAD - E
    src_p = jnp.concatenate([src, jnp.zeros((npad,), jnp.int32)])
    dst_p = jnp.concatenate([dst, jnp.full((npad,), (WT - 1) * 512, jnp.int32)])
    typ_p = jnp.concatenate([typ, jnp.zeros((npad,), jnp.int32)])

    spk_emb = _spk_emb(speaker.astype(jnp.int32).reshape(N, 1), spk_table)
    x = jnp.concatenate([encoding, spk_emb], axis=1)

    den, bk, lrk, hist = _sc_edge1(dst_p, typ_p)
    invc = _invc(den)

    hist2 = hist.reshape(NW, NBKT)
    cnt = jnp.sum(hist2, axis=0)
    caps = ((cnt + 127) // 128) * 128
    caps = jnp.where(jnp.arange(NBKT, dtype=jnp.int32) % R == 0,
                     jnp.maximum(caps, 128), caps)
    off = jnp.concatenate([jnp.zeros((1,), jnp.int32),
                           jnp.cumsum(caps)]).astype(jnp.int32)
    ub = (off[:-1] + cnt).astype(jnp.int32)
    base2 = (off[:-1][None, :]
             + (jnp.cumsum(hist2, axis=0) - hist2)).astype(jnp.int32)
    blk_starts = jnp.arange(NBLK, dtype=jnp.int32) * 128
    bkt_of_blk = (jnp.searchsorted(off[1:], blk_starts, side="right")
                  .astype(jnp.int32))
    bkt_of_blk = jnp.minimum(bkt_of_blk, NBKT - 1)
    ub_of_blk = ub[bkt_of_blk]

    pos = _pos(bk, lrk, base2)
    ed_s = _sc_permute(src_p, dst_p, pos.reshape(E_PAD // 128, 128))
    ed_s2d = ed_s.reshape(NBLK, 1, 128)

    invc3 = jnp.pad(invc[:N * R].reshape(N, R),
                    ((0, WT * 512 - N), (0, 0))).reshape(WT, 512, R)
    wr = _wr_all(comp, bases)

    xg = _sc_rowgather(x, ed_s)
    agg = _agg1(bkt_of_blk, ub_of_blk, xg, ed_s2d, wr, invc3)
    rgcn = agg.reshape(WT * 512, D)[:N]

    z, h1r = _mid(rgcn, x, root_w, rgcn_bias, gc_w_rel, gc_w_root)

    zg = _sc_rowgather(z, ed_s)
    aggz4 = _agg2(bkt_of_blk, ub_of_blk, zg, ed_s2d)
    aggz = aggz4.reshape(WT * 512, D)[:N]

    pred = _decoder(x, aggz, h1r, gc_bias, W1, b1, W2, b2, W3, b3)
    return pred.reshape(N)
